# baseline (speedup 1.0000x reference)
import jax
import jax.numpy as jnp
from jax import lax
from jax.experimental import pallas as pl
from jax.experimental.pallas import tpu as pltpu
from jax.experimental.pallas import tpu_sc as plsc

N = 10000
E = 320000
NC = 2
NS = 16
NW = NC * NS
EPW = E // NW
NBUF = 5
DEG_CHUNK = 80


def _chunk_for(din):
    return 40 if din >= 128 else 80
ROW_STRIDE = 624
ROWS_PT = 640
DEG_W = 16

_MESH = plsc.VectorSubcoreMesh(core_axis_name="c", subcore_axis_name="s")
_SC_PARAMS = pltpu.CompilerParams(use_tc_tiling_on_sc=False)

BLK = 400
GRID = N // BLK


def _spmm_call(din):
    chunk = _chunk_for(din)
    nchunk = EPW // chunk

    def body(src_hbm, dst_hbm, xn_hbm, zeros_hbm, out_hbm,
             src_v, dst_v, rows_v, acc, *sems):
        gsem, ssem = sems[:NBUF], sems[NBUF:]
        c = lax.axis_index("c")
        s = lax.axis_index("s")
        wid = s * NC + c
        pltpu.sync_copy(zeros_hbm, acc.at[pl.ds(s * ROW_STRIDE, ROWS_PT), :])
        pltpu.sync_copy(src_hbm.at[wid], src_v)
        pltpu.sync_copy(dst_hbm.at[wid], dst_v)
        plsc.subcore_barrier()

        for b in range(NBUF):
            pltpu.async_copy(xn_hbm.at[src_v.at[b]], rows_v.at[b], gsem[b])

        def outer(i, carry):
            for b in range(NBUF):
                j = i * NBUF + b
                pb = (b - 1) % NBUF
                pltpu.make_async_copy(
                    xn_hbm.at[src_v.at[j]], rows_v.at[b], gsem[b]).wait()
                pltpu.async_copy(rows_v.at[b], acc.at[dst_v.at[j]], ssem[b],
                                 add=True)
                jp = j - 1 + NBUF

                @pl.when((j >= 1) & (jp < nchunk))
                def _():
                    pltpu.make_async_copy(
                        rows_v.at[pb], acc.at[dst_v.at[jp - NBUF]],
                        ssem[pb]).wait()
                    pltpu.async_copy(
                        xn_hbm.at[src_v.at[jp]], rows_v.at[pb], gsem[pb])
            return carry

        lax.fori_loop(0, nchunk // NBUF, outer, 0)
        for b in range(NBUF):
            jj = nchunk - NBUF + b
            pltpu.make_async_copy(
                rows_v.at[jj % NBUF], acc.at[dst_v.at[jj]],
                ssem[jj % NBUF]).wait()
        plsc.subcore_barrier()
        row0 = s * ROW_STRIDE
        pltpu.sync_copy(acc.at[pl.ds(row0, ROWS_PT), :],
                        out_hbm.at[pl.ds(c * N + row0, ROWS_PT), :])

    return pl.kernel(
        body,
        out_type=jax.ShapeDtypeStruct((NC * N, din), jnp.float32),
        mesh=_MESH,
        compiler_params=_SC_PARAMS,
        scratch_types=[
            pltpu.VMEM((nchunk, chunk), jnp.int32),
            pltpu.VMEM((nchunk, chunk), jnp.int32),
            pltpu.VMEM((NBUF, chunk, din), jnp.float32),
            pltpu.VMEM_SHARED((N, din), jnp.float32),
        ] + [pltpu.SemaphoreType.DMA] * (2 * NBUF),
    )


def _degree_call():
    def body(dst_hbm, ones_hbm, zeros_hbm, out_hbm, dst_v, ones_v, acc, sem):
        c = lax.axis_index("c")
        s = lax.axis_index("s")
        wid = s * NC + c
        pltpu.sync_copy(zeros_hbm, acc.at[pl.ds(s * ROW_STRIDE, ROWS_PT), :])
        pltpu.sync_copy(ones_hbm, ones_v)
        pltpu.sync_copy(dst_hbm.at[wid], dst_v)
        plsc.subcore_barrier()

        def step(i, carry):
            for b in range(NBUF):
                pltpu.async_copy(ones_v, acc.at[dst_v.at[i * NBUF + b]], sem,
                                 add=True)
            for b in range(NBUF):
                pltpu.make_async_copy(
                    ones_v, acc.at[dst_v.at[i * NBUF + b]], sem).wait()
            return carry

        lax.fori_loop(0, (EPW // DEG_CHUNK) // NBUF, step, 0)
        plsc.subcore_barrier()
        row0 = s * ROW_STRIDE
        pltpu.sync_copy(acc.at[pl.ds(row0, ROWS_PT), :],
                        out_hbm.at[pl.ds(c * N + row0, ROWS_PT), :])

    return pl.kernel(
        body,
        out_type=jax.ShapeDtypeStruct((NC * N, DEG_W), jnp.float32),
        mesh=_MESH,
        compiler_params=_SC_PARAMS,
        scratch_types=[
            pltpu.VMEM((EPW // DEG_CHUNK, DEG_CHUNK), jnp.int32),
            pltpu.VMEM((DEG_CHUNK, DEG_W), jnp.float32),
            pltpu.VMEM_SHARED((N, DEG_W), jnp.float32),
            pltpu.SemaphoreType.DMA,
        ],
    )


def _stage0_body(dp0_ref, dp1_ref, feat_ref, norm_ref, xn_ref):
    deg = dp0_ref[:, :1] + dp1_ref[:, :1]
    nrm = lax.rsqrt(jnp.maximum(deg, 1.0))
    norm_ref[:] = nrm
    xn_ref[:] = feat_ref[:] * nrm


def _stage0(dp, features):
    return pl.pallas_call(
        _stage0_body,
        grid=(GRID,),
        in_specs=[
            pl.BlockSpec((BLK, DEG_W), lambda i: (i, 0)),
            pl.BlockSpec((BLK, DEG_W), lambda i: (i + GRID, 0)),
            pl.BlockSpec((BLK, features.shape[1]), lambda i: (i, 0)),
        ],
        out_specs=[
            pl.BlockSpec((BLK, 1), lambda i: (i, 0)),
            pl.BlockSpec((BLK, features.shape[1]), lambda i: (i, 0)),
        ],
        out_shape=[
            jax.ShapeDtypeStruct((N, 1), jnp.float32),
            jax.ShapeDtypeStruct((N, features.shape[1]), jnp.float32),
        ],
    )(dp, dp, features)


def _mid_body(p1a_ref, p1b_ref, norm_ref, xn_ref):
    nrm = norm_ref[:]
    xn_ref[:] = -(nrm * nrm) * (p1a_ref[:] + p1b_ref[:])


def _mid(p1, norm, din):
    return pl.pallas_call(
        _mid_body,
        grid=(GRID,),
        in_specs=[
            pl.BlockSpec((BLK, din), lambda i: (i, 0)),
            pl.BlockSpec((BLK, din), lambda i: (i + GRID, 0)),
            pl.BlockSpec((BLK, 1), lambda i: (i, 0)),
        ],
        out_specs=pl.BlockSpec((BLK, din), lambda i: (i, 0)),
        out_shape=jax.ShapeDtypeStruct((N, din), jnp.float32),
    )(p1, p1, norm)


def _layer_body(x_ref, p1a_ref, p1b_ref, p2a_ref, p2b_ref, norm_ref,
                wa_ref, wb_ref, wc_ref, y_ref, xn_ref=None):
    nrm = norm_ref[:]
    x = x_ref[:]
    x1 = -(nrm * (p1a_ref[:] + p1b_ref[:]))
    x2 = -2.0 * (nrm * (p2a_ref[:] + p2b_ref[:])) - x
    acc = jnp.dot(x, wa_ref[:], preferred_element_type=jnp.float32)
    acc = acc + jnp.dot(x1, wb_ref[:], preferred_element_type=jnp.float32)
    acc = acc + jnp.dot(x2, wc_ref[:], preferred_element_type=jnp.float32)
    y = jnp.maximum(acc, 0.0)
    y_ref[:] = y
    if xn_ref is not None:
        xn_ref[:] = y * nrm


def _layer(x, p1, p2, norm, W, last):
    din = x.shape[1]
    dout = W.shape[1]
    wa, wb, wc = W[:din], W[din:2 * din], W[2 * din:]
    dcol = lambda i: (i, 0)
    dcol2 = lambda i: (i + GRID, 0)
    in_specs = [
        pl.BlockSpec((BLK, din), dcol),
        pl.BlockSpec((BLK, din), dcol),
        pl.BlockSpec((BLK, din), dcol2),
        pl.BlockSpec((BLK, din), dcol),
        pl.BlockSpec((BLK, din), dcol2),
        pl.BlockSpec((BLK, 1), dcol),
        pl.BlockSpec((din, dout), lambda i: (0, 0)),
        pl.BlockSpec((din, dout), lambda i: (0, 0)),
        pl.BlockSpec((din, dout), lambda i: (0, 0)),
    ]
    if last:
        out_specs = pl.BlockSpec((BLK, dout), dcol)
        out_shape = jax.ShapeDtypeStruct((N, dout), jnp.float32)
        fn = _layer_body
    else:
        out_specs = [pl.BlockSpec((BLK, dout), dcol)] * 2
        out_shape = [jax.ShapeDtypeStruct((N, dout), jnp.float32)] * 2
        fn = lambda *refs: _layer_body(*refs[:-1], xn_ref=refs[-1])
    return pl.pallas_call(
        fn,
        grid=(GRID,),
        in_specs=in_specs,
        out_specs=out_specs,
        out_shape=out_shape,
    )(x, p1, p1, p2, p2, norm, wa, wb, wc)


@jax.jit
def kernel(edge_index, features, W0, W1, W2):
    src_flat = edge_index[0].astype(jnp.int32)
    dst_flat = edge_index[1].astype(jnp.int32)
    dst_deg = dst_flat.reshape(NW, EPW // DEG_CHUNK, DEG_CHUNK)
    features = features.astype(jnp.float32)

    ones_rows = jnp.ones((DEG_CHUNK, DEG_W), jnp.float32)
    zeros_deg = jnp.zeros((ROWS_PT, DEG_W), jnp.float32)

    dp = _degree_call()(dst_deg, ones_rows, zeros_deg)
    norm, xn = _stage0(dp, features)

    x = features
    for W, last in ((W0, False), (W1, False), (W2, True)):
        din = x.shape[1]
        zeros_blk = jnp.zeros((ROWS_PT, din), jnp.float32)
        chunk = _chunk_for(din)
        src = src_flat.reshape(NW, EPW // chunk, chunk)
        dst = dst_flat.reshape(NW, EPW // chunk, chunk)
        spmm = _spmm_call(din)
        p1 = spmm(src, dst, xn, zeros_blk)
        xn1 = _mid(p1, norm, din)
        p2 = spmm(src, dst, xn1, zeros_blk)
        if last:
            return _layer(x, p1, p2, norm, W, last=True)
        x, xn = _layer(x, p1, p2, norm, W, last=False)


# speedup vs baseline: 1.0158x; 1.0158x over previous
import jax
import jax.numpy as jnp
from jax import lax
from jax.experimental import pallas as pl
from jax.experimental.pallas import tpu as pltpu
from jax.experimental.pallas import tpu_sc as plsc

N = 10000
E = 320000
NC = 2
NS = 16
NW = NC * NS
EPW = E // NW
NBUF = 5
DEG_CHUNK = 80


def _chunk_for(din):
    return (80, 3) if din >= 128 else (80, 5)
ROW_STRIDE = 624
ROWS_PT = 640
DEG_W = 16

_MESH = plsc.VectorSubcoreMesh(core_axis_name="c", subcore_axis_name="s")
_SC_PARAMS = pltpu.CompilerParams(use_tc_tiling_on_sc=False)

BLK = 400
GRID = N // BLK


def _spmm_call(din):
    chunk, nbuf = _chunk_for(din)
    nchunk = EPW // chunk

    def body(src_hbm, dst_hbm, xn_hbm, zeros_hbm, out_hbm,
             src_v, dst_v, rows_v, acc, *sems):
        gsem, ssem = sems[:nbuf], sems[nbuf]
        c = lax.axis_index("c")
        s = lax.axis_index("s")
        wid = s * NC + c
        pltpu.sync_copy(zeros_hbm, acc.at[pl.ds(s * ROW_STRIDE, ROWS_PT), :])
        pltpu.sync_copy(src_hbm.at[wid], src_v)
        pltpu.sync_copy(dst_hbm.at[wid], dst_v)
        plsc.subcore_barrier()

        for b in range(nbuf):
            pltpu.async_copy(xn_hbm.at[src_v.at[b]], rows_v.at[b], gsem[b])

        def outer(i, carry):
            for b in range(nbuf):
                j = i * nbuf + b
                pltpu.make_async_copy(
                    xn_hbm.at[src_v.at[j]], rows_v.at[b], gsem[b]).wait()
                pltpu.async_copy(
                    rows_v.at[b], acc.at[dst_v.at[j]], ssem, add=True).wait()
                nxt = j + nbuf

                @pl.when(nxt < nchunk)
                def _():
                    pltpu.async_copy(
                        xn_hbm.at[src_v.at[nxt]], rows_v.at[b], gsem[b])
            return carry

        lax.fori_loop(0, nchunk // nbuf, outer, 0)
        for j in range((nchunk // nbuf) * nbuf, nchunk):
            b = j % nbuf
            pltpu.make_async_copy(
                xn_hbm.at[src_v.at[j]], rows_v.at[b], gsem[b]).wait()
            pltpu.async_copy(
                rows_v.at[b], acc.at[dst_v.at[j]], ssem, add=True).wait()
        plsc.subcore_barrier()
        row0 = s * ROW_STRIDE
        pltpu.sync_copy(acc.at[pl.ds(row0, ROWS_PT), :],
                        out_hbm.at[pl.ds(c * N + row0, ROWS_PT), :])

    return pl.kernel(
        body,
        out_type=jax.ShapeDtypeStruct((NC * N, din), jnp.float32),
        mesh=_MESH,
        compiler_params=_SC_PARAMS,
        scratch_types=[
            pltpu.VMEM((nchunk, chunk), jnp.int32),
            pltpu.VMEM((nchunk, chunk), jnp.int32),
            pltpu.VMEM((nbuf, chunk, din), jnp.float32),
            pltpu.VMEM_SHARED((N, din), jnp.float32),
        ] + [pltpu.SemaphoreType.DMA] * (nbuf + 1),
    )


def _degree_call():
    def body(dst_hbm, ones_hbm, zeros_hbm, out_hbm, dst_v, ones_v, acc, sem):
        c = lax.axis_index("c")
        s = lax.axis_index("s")
        wid = s * NC + c
        pltpu.sync_copy(zeros_hbm, acc.at[pl.ds(s * ROW_STRIDE, ROWS_PT), :])
        pltpu.sync_copy(ones_hbm, ones_v)
        pltpu.sync_copy(dst_hbm.at[wid], dst_v)
        plsc.subcore_barrier()

        def step(i, carry):
            for b in range(NBUF):
                pltpu.async_copy(ones_v, acc.at[dst_v.at[i * NBUF + b]], sem,
                                 add=True)
            for b in range(NBUF):
                pltpu.make_async_copy(
                    ones_v, acc.at[dst_v.at[i * NBUF + b]], sem).wait()
            return carry

        lax.fori_loop(0, (EPW // DEG_CHUNK) // NBUF, step, 0)
        plsc.subcore_barrier()
        row0 = s * ROW_STRIDE
        pltpu.sync_copy(acc.at[pl.ds(row0, ROWS_PT), :],
                        out_hbm.at[pl.ds(c * N + row0, ROWS_PT), :])

    return pl.kernel(
        body,
        out_type=jax.ShapeDtypeStruct((NC * N, DEG_W), jnp.float32),
        mesh=_MESH,
        compiler_params=_SC_PARAMS,
        scratch_types=[
            pltpu.VMEM((EPW // DEG_CHUNK, DEG_CHUNK), jnp.int32),
            pltpu.VMEM((DEG_CHUNK, DEG_W), jnp.float32),
            pltpu.VMEM_SHARED((N, DEG_W), jnp.float32),
            pltpu.SemaphoreType.DMA,
        ],
    )


def _stage0_body(dp0_ref, dp1_ref, feat_ref, norm_ref, xn_ref):
    deg = dp0_ref[:, :1] + dp1_ref[:, :1]
    nrm = lax.rsqrt(jnp.maximum(deg, 1.0))
    norm_ref[:] = nrm
    xn_ref[:] = feat_ref[:] * nrm


def _stage0(dp, features):
    return pl.pallas_call(
        _stage0_body,
        grid=(GRID,),
        in_specs=[
            pl.BlockSpec((BLK, DEG_W), lambda i: (i, 0)),
            pl.BlockSpec((BLK, DEG_W), lambda i: (i + GRID, 0)),
            pl.BlockSpec((BLK, features.shape[1]), lambda i: (i, 0)),
        ],
        out_specs=[
            pl.BlockSpec((BLK, 1), lambda i: (i, 0)),
            pl.BlockSpec((BLK, features.shape[1]), lambda i: (i, 0)),
        ],
        out_shape=[
            jax.ShapeDtypeStruct((N, 1), jnp.float32),
            jax.ShapeDtypeStruct((N, features.shape[1]), jnp.float32),
        ],
    )(dp, dp, features)


def _mid_body(p1a_ref, p1b_ref, norm_ref, xn_ref):
    nrm = norm_ref[:]
    xn_ref[:] = -(nrm * nrm) * (p1a_ref[:] + p1b_ref[:])


def _mid(p1, norm, din):
    return pl.pallas_call(
        _mid_body,
        grid=(GRID,),
        in_specs=[
            pl.BlockSpec((BLK, din), lambda i: (i, 0)),
            pl.BlockSpec((BLK, din), lambda i: (i + GRID, 0)),
            pl.BlockSpec((BLK, 1), lambda i: (i, 0)),
        ],
        out_specs=pl.BlockSpec((BLK, din), lambda i: (i, 0)),
        out_shape=jax.ShapeDtypeStruct((N, din), jnp.float32),
    )(p1, p1, norm)


def _layer_body(x_ref, p1a_ref, p1b_ref, p2a_ref, p2b_ref, norm_ref,
                wa_ref, wb_ref, wc_ref, y_ref, xn_ref=None):
    nrm = norm_ref[:]
    x = x_ref[:]
    x1 = -(nrm * (p1a_ref[:] + p1b_ref[:]))
    x2 = -2.0 * (nrm * (p2a_ref[:] + p2b_ref[:])) - x
    acc = jnp.dot(x, wa_ref[:], preferred_element_type=jnp.float32)
    acc = acc + jnp.dot(x1, wb_ref[:], preferred_element_type=jnp.float32)
    acc = acc + jnp.dot(x2, wc_ref[:], preferred_element_type=jnp.float32)
    y = jnp.maximum(acc, 0.0)
    y_ref[:] = y
    if xn_ref is not None:
        xn_ref[:] = y * nrm


def _layer(x, p1, p2, norm, W, last):
    din = x.shape[1]
    dout = W.shape[1]
    wa, wb, wc = W[:din], W[din:2 * din], W[2 * din:]
    dcol = lambda i: (i, 0)
    dcol2 = lambda i: (i + GRID, 0)
    in_specs = [
        pl.BlockSpec((BLK, din), dcol),
        pl.BlockSpec((BLK, din), dcol),
        pl.BlockSpec((BLK, din), dcol2),
        pl.BlockSpec((BLK, din), dcol),
        pl.BlockSpec((BLK, din), dcol2),
        pl.BlockSpec((BLK, 1), dcol),
        pl.BlockSpec((din, dout), lambda i: (0, 0)),
        pl.BlockSpec((din, dout), lambda i: (0, 0)),
        pl.BlockSpec((din, dout), lambda i: (0, 0)),
    ]
    if last:
        out_specs = pl.BlockSpec((BLK, dout), dcol)
        out_shape = jax.ShapeDtypeStruct((N, dout), jnp.float32)
        fn = _layer_body
    else:
        out_specs = [pl.BlockSpec((BLK, dout), dcol)] * 2
        out_shape = [jax.ShapeDtypeStruct((N, dout), jnp.float32)] * 2
        fn = lambda *refs: _layer_body(*refs[:-1], xn_ref=refs[-1])
    return pl.pallas_call(
        fn,
        grid=(GRID,),
        in_specs=in_specs,
        out_specs=out_specs,
        out_shape=out_shape,
    )(x, p1, p1, p2, p2, norm, wa, wb, wc)


@jax.jit
def kernel(edge_index, features, W0, W1, W2):
    src_flat = edge_index[0].astype(jnp.int32)
    dst_flat = edge_index[1].astype(jnp.int32)
    dst_deg = dst_flat.reshape(NW, EPW // DEG_CHUNK, DEG_CHUNK)
    features = features.astype(jnp.float32)

    ones_rows = jnp.ones((DEG_CHUNK, DEG_W), jnp.float32)
    zeros_deg = jnp.zeros((ROWS_PT, DEG_W), jnp.float32)

    dp = _degree_call()(dst_deg, ones_rows, zeros_deg)
    norm, xn = _stage0(dp, features)

    x = features
    for W, last in ((W0, False), (W1, False), (W2, True)):
        din = x.shape[1]
        zeros_blk = jnp.zeros((ROWS_PT, din), jnp.float32)
        chunk, _ = _chunk_for(din)
        src = src_flat.reshape(NW, EPW // chunk, chunk)
        dst = dst_flat.reshape(NW, EPW // chunk, chunk)
        spmm = _spmm_call(din)
        p1 = spmm(src, dst, xn, zeros_blk)
        xn1 = _mid(p1, norm, din)
        p2 = spmm(src, dst, xn1, zeros_blk)
        if last:
            return _layer(x, p1, p2, norm, W, last=True)
        x, xn = _layer(x, p1, p2, norm, W, last=False)


# speedup vs baseline: 1.1449x; 1.1271x over previous
import jax
import jax.numpy as jnp
from jax import lax
from jax.experimental import pallas as pl
from jax.experimental.pallas import tpu as pltpu
from jax.experimental.pallas import tpu_sc as plsc

N = 10000
E = 320000
NC = 2
NS = 16
NW = NC * NS
EPW = E // NW
NBUF = 5
DEG_CHUNK = 80


def _chunk_for(din):
    return (40, 5) if din >= 128 else (80, 5)
ROW_STRIDE = 624
ROWS_PT = 640
DEG_W = 16

_MESH = plsc.VectorSubcoreMesh(core_axis_name="c", subcore_axis_name="s")
_SC_PARAMS_LINEAR = pltpu.CompilerParams(use_tc_tiling_on_sc=False)
_SC_PARAMS_TILED = pltpu.CompilerParams(use_tc_tiling_on_sc=True)

BLK = 2000
GRID = N // BLK


def _spmm_call(din):
    chunk, nbuf = _chunk_for(din)
    nchunk = EPW // chunk

    def body(src_hbm, dst_hbm, xn_hbm, zeros_hbm, out_hbm,
             src_v, dst_v, rows_v, acc, *sems):
        gsem, ssem = sems[:nbuf], sems[nbuf]
        c = lax.axis_index("c")
        s = lax.axis_index("s")
        wid = s * NC + c
        pltpu.sync_copy(zeros_hbm, acc.at[pl.ds(s * ROW_STRIDE, ROWS_PT), :])
        pltpu.sync_copy(src_hbm.at[wid], src_v)
        pltpu.sync_copy(dst_hbm.at[wid], dst_v)
        plsc.subcore_barrier()

        for b in range(nbuf):
            pltpu.async_copy(xn_hbm.at[src_v.at[b]], rows_v.at[b], gsem[b])

        def outer(i, carry):
            for b in range(nbuf):
                j = i * nbuf + b
                pltpu.make_async_copy(
                    xn_hbm.at[src_v.at[j]], rows_v.at[b], gsem[b]).wait()
                pltpu.async_copy(
                    rows_v.at[b], acc.at[dst_v.at[j]], ssem, add=True).wait()
                nxt = j + nbuf

                @pl.when(nxt < nchunk)
                def _():
                    pltpu.async_copy(
                        xn_hbm.at[src_v.at[nxt]], rows_v.at[b], gsem[b])
            return carry

        lax.fori_loop(0, nchunk // nbuf, outer, 0)
        for j in range((nchunk // nbuf) * nbuf, nchunk):
            b = j % nbuf
            pltpu.make_async_copy(
                xn_hbm.at[src_v.at[j]], rows_v.at[b], gsem[b]).wait()
            pltpu.async_copy(
                rows_v.at[b], acc.at[dst_v.at[j]], ssem, add=True).wait()
        plsc.subcore_barrier()
        row0 = s * ROW_STRIDE
        pltpu.sync_copy(acc.at[pl.ds(row0, ROWS_PT), :],
                        out_hbm.at[pl.ds(c * N + row0, ROWS_PT), :])

    return pl.kernel(
        body,
        out_type=jax.ShapeDtypeStruct((NC * N, din), jnp.float32),
        mesh=_MESH,
        compiler_params=_SC_PARAMS_LINEAR,
        scratch_types=[
            pltpu.VMEM((nchunk, chunk), jnp.int32),
            pltpu.VMEM((nchunk, chunk), jnp.int32),
            pltpu.VMEM((nbuf, chunk, din), jnp.float32),
            pltpu.VMEM_SHARED((N, din), jnp.float32),
        ] + [pltpu.SemaphoreType.DMA] * (nbuf + 1),
    )


def _degree_call():
    def body(dst_hbm, ones_hbm, zeros_hbm, out_hbm, dst_v, ones_v, acc, sem):
        c = lax.axis_index("c")
        s = lax.axis_index("s")
        wid = s * NC + c
        pltpu.sync_copy(zeros_hbm, acc.at[pl.ds(s * ROW_STRIDE, ROWS_PT), :])
        pltpu.sync_copy(ones_hbm, ones_v)
        pltpu.sync_copy(dst_hbm.at[wid], dst_v)
        plsc.subcore_barrier()

        def step(i, carry):
            for b in range(NBUF):
                pltpu.async_copy(ones_v, acc.at[dst_v.at[i * NBUF + b]], sem,
                                 add=True)
            for b in range(NBUF):
                pltpu.make_async_copy(
                    ones_v, acc.at[dst_v.at[i * NBUF + b]], sem).wait()
            return carry

        lax.fori_loop(0, (EPW // DEG_CHUNK) // NBUF, step, 0)
        plsc.subcore_barrier()
        row0 = s * ROW_STRIDE
        pltpu.sync_copy(acc.at[pl.ds(row0, ROWS_PT), :],
                        out_hbm.at[pl.ds(c * N + row0, ROWS_PT), :])

    return pl.kernel(
        body,
        out_type=jax.ShapeDtypeStruct((NC * N, DEG_W), jnp.float32),
        mesh=_MESH,
        compiler_params=_SC_PARAMS_LINEAR,
        scratch_types=[
            pltpu.VMEM((EPW // DEG_CHUNK, DEG_CHUNK), jnp.int32),
            pltpu.VMEM((DEG_CHUNK, DEG_W), jnp.float32),
            pltpu.VMEM_SHARED((N, DEG_W), jnp.float32),
            pltpu.SemaphoreType.DMA,
        ],
    )


def _stage0_body(dp0_ref, dp1_ref, feat_ref, norm_ref, xn_ref):
    deg = dp0_ref[:, :1] + dp1_ref[:, :1]
    nrm = lax.rsqrt(jnp.maximum(deg, 1.0))
    norm_ref[:] = nrm
    xn_ref[:] = feat_ref[:] * nrm


def _stage0(dp, features):
    return pl.pallas_call(
        _stage0_body,
        grid=(GRID,),
        in_specs=[
            pl.BlockSpec((BLK, DEG_W), lambda i: (i, 0)),
            pl.BlockSpec((BLK, DEG_W), lambda i: (i + GRID, 0)),
            pl.BlockSpec((BLK, features.shape[1]), lambda i: (i, 0)),
        ],
        out_specs=[
            pl.BlockSpec((BLK, 1), lambda i: (i, 0)),
            pl.BlockSpec((BLK, features.shape[1]), lambda i: (i, 0)),
        ],
        out_shape=[
            jax.ShapeDtypeStruct((N, 1), jnp.float32),
            jax.ShapeDtypeStruct((N, features.shape[1]), jnp.float32),
        ],
    )(dp, dp, features)


def _mid_body(p1a_ref, p1b_ref, norm_ref, xn_ref):
    nrm = norm_ref[:]
    xn_ref[:] = -(nrm * nrm) * (p1a_ref[:] + p1b_ref[:])


def _mid(p1, norm, din):
    return pl.pallas_call(
        _mid_body,
        grid=(GRID,),
        in_specs=[
            pl.BlockSpec((BLK, din), lambda i: (i, 0)),
            pl.BlockSpec((BLK, din), lambda i: (i + GRID, 0)),
            pl.BlockSpec((BLK, 1), lambda i: (i, 0)),
        ],
        out_specs=pl.BlockSpec((BLK, din), lambda i: (i, 0)),
        out_shape=jax.ShapeDtypeStruct((N, din), jnp.float32),
    )(p1, p1, norm)


def _layer_body(x_ref, p1a_ref, p1b_ref, p2a_ref, p2b_ref, norm_ref,
                wa_ref, wb_ref, wc_ref, y_ref, xn_ref=None):
    nrm = norm_ref[:]
    x = x_ref[:]
    x1 = -(nrm * (p1a_ref[:] + p1b_ref[:]))
    x2 = -2.0 * (nrm * (p2a_ref[:] + p2b_ref[:])) - x
    acc = jnp.dot(x, wa_ref[:], preferred_element_type=jnp.float32)
    acc = acc + jnp.dot(x1, wb_ref[:], preferred_element_type=jnp.float32)
    acc = acc + jnp.dot(x2, wc_ref[:], preferred_element_type=jnp.float32)
    y = jnp.maximum(acc, 0.0)
    y_ref[:] = y
    if xn_ref is not None:
        xn_ref[:] = y * nrm


def _layer(x, p1, p2, norm, W, last):
    din = x.shape[1]
    dout = W.shape[1]
    wa, wb, wc = W[:din], W[din:2 * din], W[2 * din:]
    dcol = lambda i: (i, 0)
    dcol2 = lambda i: (i + GRID, 0)
    in_specs = [
        pl.BlockSpec((BLK, din), dcol),
        pl.BlockSpec((BLK, din), dcol),
        pl.BlockSpec((BLK, din), dcol2),
        pl.BlockSpec((BLK, din), dcol),
        pl.BlockSpec((BLK, din), dcol2),
        pl.BlockSpec((BLK, 1), dcol),
        pl.BlockSpec((din, dout), lambda i: (0, 0)),
        pl.BlockSpec((din, dout), lambda i: (0, 0)),
        pl.BlockSpec((din, dout), lambda i: (0, 0)),
    ]
    if last:
        out_specs = pl.BlockSpec((BLK, dout), dcol)
        out_shape = jax.ShapeDtypeStruct((N, dout), jnp.float32)
        fn = _layer_body
    else:
        out_specs = [pl.BlockSpec((BLK, dout), dcol)] * 2
        out_shape = [jax.ShapeDtypeStruct((N, dout), jnp.float32)] * 2
        fn = lambda *refs: _layer_body(*refs[:-1], xn_ref=refs[-1])
    return pl.pallas_call(
        fn,
        grid=(GRID,),
        in_specs=in_specs,
        out_specs=out_specs,
        out_shape=out_shape,
    )(x, p1, p1, p2, p2, norm, wa, wb, wc)


@jax.jit
def kernel(edge_index, features, W0, W1, W2):
    src_flat = edge_index[0].astype(jnp.int32)
    dst_flat = edge_index[1].astype(jnp.int32)
    dst_deg = dst_flat.reshape(NW, EPW // DEG_CHUNK, DEG_CHUNK)
    features = features.astype(jnp.float32)

    ones_rows = jnp.ones((DEG_CHUNK, DEG_W), jnp.float32)
    zeros_deg = jnp.zeros((ROWS_PT, DEG_W), jnp.float32)

    dp = _degree_call()(dst_deg, ones_rows, zeros_deg)
    norm, xn = _stage0(dp, features)

    x = features
    for W, last in ((W0, False), (W1, False), (W2, True)):
        din = x.shape[1]
        zeros_blk = jnp.zeros((ROWS_PT, din), jnp.float32)
        chunk, _ = _chunk_for(din)
        src = src_flat.reshape(NW, EPW // chunk, chunk)
        dst = dst_flat.reshape(NW, EPW // chunk, chunk)
        spmm = _spmm_call(din)
        p1 = spmm(src, dst, xn, zeros_blk)
        xn1 = _mid(p1, norm, din)
        p2 = spmm(src, dst, xn1, zeros_blk)
        if last:
            return _layer(x, p1, p2, norm, W, last=True)
        x, xn = _layer(x, p1, p2, norm, W, last=False)
